# baseline (speedup 1.0000x reference)
import functools

import jax
import jax.numpy as jnp
from jax import lax
from jax.experimental import pallas as pl
from jax.experimental.pallas import tpu as pltpu
from jax.experimental.pallas import tpu_sc as plsc

N_NODES = 10000
N_PAD = 10240
N_EDGES = 320000
D = 128
NC = 2
NS = 16
NW = NC * NS
EPT = N_EDGES // NW
C = 80
NCHUNK = EPT // C
RPT = N_PAD // NS

_mesh = plsc.VectorSubcoreMesh(core_axis_name="c", subcore_axis_name="s")
_f32 = jnp.float32


@functools.partial(
    pl.kernel,
    out_type=jax.ShapeDtypeStruct((NW, N_PAD), _f32),
    mesh=_mesh,
    scratch_types=[
        pltpu.VMEM((EPT,), jnp.int32),
        pltpu.VMEM((N_PAD,), _f32),
    ],
    compiler_params=pltpu.CompilerParams(needs_layout_passes=False),
)
def _deg_kernel(edge_hbm, deg_out, didx, deg_v):
    cid = lax.axis_index("c")
    sid = lax.axis_index("s")
    wid = cid * NS + sid
    pltpu.sync_copy(edge_hbm.at[pl.ds(N_EDGES + wid * EPT, EPT)], didx)

    def _zero(i, _):
        deg_v[pl.ds(i * 16, 16)] = jnp.zeros((16,), _f32)
        return 0

    lax.fori_loop(0, N_PAD // 16, _zero, 0)

    ones = jnp.ones((16,), _f32)

    def _acc(i, _):
        plsc.addupdate_scatter(deg_v, [didx[pl.ds(i * 16, 16)]], ones)
        return 0

    lax.fori_loop(0, EPT // 16, _acc, 0)
    pltpu.sync_copy(deg_v, deg_out.at[wid])


@functools.partial(
    pl.kernel,
    out_type=jax.ShapeDtypeStruct((NC, N_PAD, D), _f32),
    mesh=_mesh,
    scratch_types=[
        pltpu.VMEM((EPT,), jnp.int32),
        pltpu.VMEM((NCHUNK, C), jnp.int32),
        pltpu.VMEM((C, D), _f32),
        pltpu.VMEM((C, D), _f32),
        pltpu.SemaphoreType.DMA,
        pltpu.SemaphoreType.DMA,
        pltpu.SemaphoreType.DMA,
        pltpu.VMEM_SHARED((N_PAD, D), _f32),
    ],
)
def _prop_kernel(tin_hbm, edge_hbm, out_hbm,
                 sidx, didx, rows_a, rows_b, sr_a, sr_b, sem_d, acc):
    cid = lax.axis_index("c")
    sid = lax.axis_index("s")
    wid = cid * NS + sid
    rbufs = (rows_a, rows_b)
    rsems = (sr_a, sr_b)
    pltpu.sync_copy(edge_hbm.at[pl.ds(wid * EPT, EPT)], sidx)

    def _drow(j, _):
        pltpu.async_copy(edge_hbm.at[pl.ds(N_EDGES + wid * EPT + j * C, C)],
                         didx.at[j], sem_d)
        return 0

    lax.fori_loop(0, NCHUNK, _drow, 0)

    def _zrow(r, _):
        for cblk in range(D // 16):
            rows_a[r, pl.ds(cblk * 16, 16)] = jnp.zeros((16,), _f32)
        return 0

    lax.fori_loop(0, C, _zrow, 0)
    for m in range(RPT // C):
        pltpu.async_copy(rows_a, acc.at[pl.ds(sid * RPT + m * C, C)], sr_a)
    for m in range(RPT // C):
        pltpu.make_async_copy(rows_a, acc.at[pl.ds(sid * RPT, C)], sr_a).wait()

    def _dwait(j, _):
        pltpu.make_async_copy(edge_hbm.at[pl.ds(0, C)], didx.at[0],
                              sem_d).wait()
        return 0

    lax.fori_loop(0, NCHUNK, _dwait, 0)

    def fire(j, p):
        pltpu.async_copy(tin_hbm.at[sidx.at[pl.ds(j * C, C)]], rbufs[p],
                         rsems[p])

    def wait_scatter(j, p):
        pltpu.make_async_copy(tin_hbm.at[sidx.at[pl.ds(0, C)]], rbufs[p],
                              rsems[p]).wait()
        pltpu.sync_copy(rbufs[p], acc.at[didx.at[j]], add=True)

    fire(0, 0)
    fire(1, 1)
    plsc.subcore_barrier()

    def _pair(i, _):
        j0 = 2 * i
        wait_scatter(j0, 0)

        @pl.when(j0 + 2 < NCHUNK)
        def _():
            fire(j0 + 2, 0)

        wait_scatter(j0 + 1, 1)

        @pl.when(j0 + 3 < NCHUNK)
        def _():
            fire(j0 + 3, 1)

        return 0

    lax.fori_loop(0, NCHUNK // 2, _pair, 0)
    wait_scatter(NCHUNK - 1, 0)
    plsc.subcore_barrier()

    for m in range(RPT // C):
        r0 = sid * RPT + m * C
        if m >= 2:
            pltpu.make_async_copy(rbufs[m % 2], out_hbm.at[cid, pl.ds(r0, C)],
                                  rsems[m % 2]).wait()
        pltpu.sync_copy(acc.at[pl.ds(r0, C)], rbufs[m % 2])
        pltpu.async_copy(rbufs[m % 2], out_hbm.at[cid, pl.ds(r0, C)],
                         rsems[m % 2])
    pltpu.make_async_copy(rows_a, out_hbm.at[cid, pl.ds(0, C)], sr_a).wait()
    pltpu.make_async_copy(rows_b, out_hbm.at[cid, pl.ds(0, C)], sr_b).wait()


def _rownorm(degp_ref):
    deg = jnp.sum(degp_ref[...], axis=0, keepdims=True)
    return jnp.maximum(deg, 1.0).T


def _scale_body(x_ref, degp_ref, o_ref):
    norm = lax.rsqrt(_rownorm(degp_ref))
    o_ref[...] = x_ref[...] * norm


def _combine_body(parts_ref, degp_ref, o_ref):
    inv = 1.0 / _rownorm(degp_ref)
    o_ref[...] = (parts_ref[0] + parts_ref[1]) * inv


def _final_body(parts_ref, degp_ref, w_ref, b_ref, o_ref):
    norm = lax.rsqrt(_rownorm(degp_ref))
    s = (parts_ref[0] + parts_ref[1]) * norm
    o_ref[...] = jnp.dot(s, w_ref[...], preferred_element_type=_f32) + b_ref[...]


_ROWS_BLK = 10240
_GRID = N_PAD // _ROWS_BLK


def _tc_scale(x, degp):
    return pl.pallas_call(
        _scale_body,
        grid=(_GRID,),
        in_specs=[
            pl.BlockSpec((_ROWS_BLK, D), lambda i: (i, 0)),
            pl.BlockSpec((NW, _ROWS_BLK), lambda i: (0, i)),
        ],
        out_specs=pl.BlockSpec((_ROWS_BLK, D), lambda i: (i, 0)),
        out_shape=jax.ShapeDtypeStruct((N_PAD, D), _f32),
    )(x, degp)


def _tc_combine(parts, degp):
    return pl.pallas_call(
        _combine_body,
        grid=(_GRID,),
        in_specs=[
            pl.BlockSpec((NC, _ROWS_BLK, D), lambda i: (0, i, 0)),
            pl.BlockSpec((NW, _ROWS_BLK), lambda i: (0, i)),
        ],
        out_specs=pl.BlockSpec((_ROWS_BLK, D), lambda i: (i, 0)),
        out_shape=jax.ShapeDtypeStruct((N_PAD, D), _f32),
    )(parts, degp)


def _tc_final(parts, degp, W, b2):
    return pl.pallas_call(
        _final_body,
        grid=(_GRID,),
        in_specs=[
            pl.BlockSpec((NC, _ROWS_BLK, D), lambda i: (0, i, 0)),
            pl.BlockSpec((NW, _ROWS_BLK), lambda i: (0, i)),
            pl.BlockSpec((D, D), lambda i: (0, 0)),
            pl.BlockSpec((1, D), lambda i: (0, 0)),
        ],
        out_specs=pl.BlockSpec((_ROWS_BLK, D), lambda i: (i, 0)),
        out_shape=jax.ShapeDtypeStruct((N_NODES, D), _f32),
    )(parts, degp, W, b2)


def kernel(inputs, edge_index, W, b):
    edge = edge_index.astype(jnp.int32).reshape(2 * N_EDGES)
    b2 = b.reshape(1, D)

    degp = _deg_kernel(edge)

    t0 = _tc_scale(inputs, degp)
    p1 = _prop_kernel(t0, edge)
    t1 = _tc_combine(p1, degp)
    p2 = _prop_kernel(t1, edge)
    h = _tc_final(p2, degp, W, b2)

    return (h, 0)


# speedup vs baseline: 1.0211x; 1.0211x over previous
import functools

import jax
import jax.numpy as jnp
from jax import lax
from jax.experimental import pallas as pl
from jax.experimental.pallas import tpu as pltpu
from jax.experimental.pallas import tpu_sc as plsc

N_NODES = 10000
N_PAD = 10240
N_EDGES = 320000
D = 128
NC = 2
NS = 16
NW = NC * NS
EPT = N_EDGES // NW
C = 80
NCHUNK = EPT // C
RPT = N_PAD // NS

_mesh = plsc.VectorSubcoreMesh(core_axis_name="c", subcore_axis_name="s")
_f32 = jnp.float32


@functools.partial(
    pl.kernel,
    out_type=jax.ShapeDtypeStruct((NW, N_PAD), _f32),
    mesh=_mesh,
    scratch_types=[
        pltpu.VMEM((EPT,), jnp.int32),
        pltpu.VMEM((N_PAD,), _f32),
    ],
    compiler_params=pltpu.CompilerParams(needs_layout_passes=False),
)
def _deg_kernel(edge_hbm, deg_out, didx, deg_v):
    cid = lax.axis_index("c")
    sid = lax.axis_index("s")
    wid = cid * NS + sid
    pltpu.sync_copy(edge_hbm.at[pl.ds(N_EDGES + wid * EPT, EPT)], didx)

    def _zero(i, _):
        deg_v[pl.ds(i * 16, 16)] = jnp.zeros((16,), _f32)
        return 0

    lax.fori_loop(0, N_PAD // 16, _zero, 0)

    ones = jnp.ones((16,), _f32)

    def _acc(i, _):
        plsc.addupdate_scatter(deg_v, [didx[pl.ds(i * 16, 16)]], ones)
        return 0

    lax.fori_loop(0, EPT // 16, _acc, 0)
    pltpu.sync_copy(deg_v, deg_out.at[wid])


@functools.partial(
    pl.kernel,
    out_type=jax.ShapeDtypeStruct((NC, N_PAD, D), _f32),
    mesh=_mesh,
    scratch_types=[
        pltpu.VMEM((EPT,), jnp.int32),
        pltpu.VMEM((NCHUNK, C), jnp.int32),
        pltpu.VMEM((C, D), _f32),
        pltpu.VMEM((C, D), _f32),
        pltpu.SemaphoreType.DMA,
        pltpu.SemaphoreType.DMA,
        pltpu.SemaphoreType.DMA,
        pltpu.VMEM_SHARED((N_PAD, D), _f32),
    ],
)
def _prop_kernel(tin_hbm, edge_hbm, out_hbm,
                 sidx, didx, rows_a, rows_b, sr_a, sr_b, sem_d, acc):
    cid = lax.axis_index("c")
    sid = lax.axis_index("s")
    wid = cid * NS + sid
    rbufs = (rows_a, rows_b)
    rsems = (sr_a, sr_b)
    pltpu.sync_copy(edge_hbm.at[pl.ds(wid * EPT, EPT)], sidx)

    def _drow(j, _):
        pltpu.async_copy(edge_hbm.at[pl.ds(N_EDGES + wid * EPT + j * C, C)],
                         didx.at[j], sem_d)
        return 0

    lax.fori_loop(0, NCHUNK, _drow, 0)

    def _zrow(r, _):
        for cblk in range(D // 16):
            rows_a[r, pl.ds(cblk * 16, 16)] = jnp.zeros((16,), _f32)
        return 0

    lax.fori_loop(0, C, _zrow, 0)
    for m in range(RPT // C):
        pltpu.async_copy(rows_a, acc.at[pl.ds(sid * RPT + m * C, C)], sr_a)
    for m in range(RPT // C):
        pltpu.make_async_copy(rows_a, acc.at[pl.ds(sid * RPT, C)], sr_a).wait()

    def _dwait(j, _):
        pltpu.make_async_copy(edge_hbm.at[pl.ds(0, C)], didx.at[0],
                              sem_d).wait()
        return 0

    lax.fori_loop(0, NCHUNK, _dwait, 0)

    def fire(j, p):
        pltpu.async_copy(tin_hbm.at[sidx.at[pl.ds(j * C, C)]], rbufs[p],
                         rsems[p])

    def wait_scatter(j, p):
        pltpu.make_async_copy(tin_hbm.at[sidx.at[pl.ds(0, C)]], rbufs[p],
                              rsems[p]).wait()
        pltpu.sync_copy(rbufs[p], acc.at[didx.at[j]], add=True)

    fire(0, 0)
    fire(1, 1)
    plsc.subcore_barrier()

    def _pair(i, _):
        j0 = 2 * i
        wait_scatter(j0, 0)

        @pl.when(j0 + 2 < NCHUNK)
        def _():
            fire(j0 + 2, 0)

        wait_scatter(j0 + 1, 1)

        @pl.when(j0 + 3 < NCHUNK)
        def _():
            fire(j0 + 3, 1)

        return 0

    lax.fori_loop(0, NCHUNK // 2, _pair, 0)
    wait_scatter(NCHUNK - 1, 0)
    plsc.subcore_barrier()

    for m in range(RPT // C):
        r0 = sid * RPT + m * C
        if m >= 2:
            pltpu.make_async_copy(rbufs[m % 2], out_hbm.at[cid, pl.ds(r0, C)],
                                  rsems[m % 2]).wait()
        pltpu.sync_copy(acc.at[pl.ds(r0, C)], rbufs[m % 2])
        pltpu.async_copy(rbufs[m % 2], out_hbm.at[cid, pl.ds(r0, C)],
                         rsems[m % 2])
    pltpu.make_async_copy(rows_a, out_hbm.at[cid, pl.ds(0, C)], sr_a).wait()
    pltpu.make_async_copy(rows_b, out_hbm.at[cid, pl.ds(0, C)], sr_b).wait()


def _rownorm(degp_ref):
    deg = jnp.sum(degp_ref[...], axis=0, keepdims=True)
    return jnp.maximum(deg, 1.0).T


def _scale_body(x_ref, degp_ref, o_ref):
    norm = lax.rsqrt(_rownorm(degp_ref))
    o_ref[...] = x_ref[...] * norm


def _combine_body(parts_ref, degp_ref, o_ref):
    inv = 1.0 / _rownorm(degp_ref)
    o_ref[...] = (parts_ref[0] + parts_ref[1]) * inv


def _final_body(parts_ref, degp_ref, w_ref, b_ref, o_ref):
    norm = lax.rsqrt(_rownorm(degp_ref))
    s = (parts_ref[0] + parts_ref[1]) * norm
    o_ref[...] = jnp.dot(s, w_ref[...], preferred_element_type=_f32) + b_ref[...]


_ROWS_BLK = 5120
_GRID = N_PAD // _ROWS_BLK


def _tc_scale(x, degp):
    return pl.pallas_call(
        _scale_body,
        grid=(_GRID,),
        in_specs=[
            pl.BlockSpec((_ROWS_BLK, D), lambda i: (i, 0)),
            pl.BlockSpec((NW, _ROWS_BLK), lambda i: (0, i)),
        ],
        out_specs=pl.BlockSpec((_ROWS_BLK, D), lambda i: (i, 0)),
        out_shape=jax.ShapeDtypeStruct((N_PAD, D), _f32),
    )(x, degp)


def _tc_combine(parts, degp):
    return pl.pallas_call(
        _combine_body,
        grid=(_GRID,),
        in_specs=[
            pl.BlockSpec((NC, _ROWS_BLK, D), lambda i: (0, i, 0)),
            pl.BlockSpec((NW, _ROWS_BLK), lambda i: (0, i)),
        ],
        out_specs=pl.BlockSpec((_ROWS_BLK, D), lambda i: (i, 0)),
        out_shape=jax.ShapeDtypeStruct((N_PAD, D), _f32),
    )(parts, degp)


def _tc_final(parts, degp, W, b2):
    return pl.pallas_call(
        _final_body,
        grid=(_GRID,),
        in_specs=[
            pl.BlockSpec((NC, _ROWS_BLK, D), lambda i: (0, i, 0)),
            pl.BlockSpec((NW, _ROWS_BLK), lambda i: (0, i)),
            pl.BlockSpec((D, D), lambda i: (0, 0)),
            pl.BlockSpec((1, D), lambda i: (0, 0)),
        ],
        out_specs=pl.BlockSpec((_ROWS_BLK, D), lambda i: (i, 0)),
        out_shape=jax.ShapeDtypeStruct((N_NODES, D), _f32),
    )(parts, degp, W, b2)


def kernel(inputs, edge_index, W, b):
    edge = edge_index.astype(jnp.int32).reshape(2 * N_EDGES)
    b2 = b.reshape(1, D)

    degp = _deg_kernel(edge)

    t0 = _tc_scale(inputs, degp)
    p1 = _prop_kernel(t0, edge)
    t1 = _tc_combine(p1, degp)
    p2 = _prop_kernel(t1, edge)
    h = _tc_final(p2, degp, W, b2)

    return (h, 0)
